# baseline (speedup 1.0000x reference)
import jax
import jax.numpy as jnp
from jax import lax
from jax.experimental import pallas as pl
from jax.experimental.pallas import tpu as pltpu
from jax.experimental.pallas import tpu_sc as plsc

_M = 1024
_D = 16
_N = 100000
_NPAD = 100352
_BN = 2048
_K = 16
_NW = 32
_QPW = _M // _NW
_C = 2048
_NCH = _NPAD // _C
_GPC = _C // 16
_GC = _C // 16
_VPC = _GPC // 16
_PADVAL = 1e18


_BM = 256


def _proxy_body(qm2_ref, s_ref, out_ref, min_ref):
  s = s_ref[...]
  sn = jnp.sum(s * s, axis=1)
  acc = lax.dot_general(qm2_ref[...], s, (((1,), (1,)), ((), ())),
                        preferred_element_type=jnp.float32)
  prox = acc + sn[None, :]
  out_ref[...] = prox
  min_ref[...] = jnp.min(prox.reshape(_BM, _BN // 16, 16), axis=-1)


def _compute_proxy(qm2, spad):
  return pl.pallas_call(
      _proxy_body,
      grid=(_M // _BM, _NPAD // _BN),
      in_specs=[
          pl.BlockSpec((_BM, _D), lambda m, n: (m, 0)),
          pl.BlockSpec((_BN, _D), lambda m, n: (n, 0)),
      ],
      out_specs=[
          pl.BlockSpec((_BM, _BN), lambda m, n: (m, n)),
          pl.BlockSpec((_BM, _BN // 16), lambda m, n: (m, n)),
      ],
      out_shape=[
          jax.ShapeDtypeStruct((_M, _NPAD), jnp.float32),
          jax.ShapeDtypeStruct((_M, _NPAD // 16), jnp.float32),
      ],
  )(qm2, spad)


def _merge16(bv, bi, cv, ci):
  cs, cis = plsc.sort_key_val(cv, ci)
  cr = lax.rev(cs, (0,))
  cir = lax.rev(cis, (0,))
  take = cr < bv
  nv = jnp.where(take, cr, bv)
  ni = jnp.where(take, cir, bi)
  return plsc.sort_key_val(nv, ni)


def _scan_chunk(pbuf, gbuf, chunk_i, carry):
  iota = lax.iota(jnp.int32, 16)
  cbase = chunk_i * _C
  out = list(carry)
  for r in range(8):
    bv, bi, thr = carry[3 * r], carry[3 * r + 1], carry[3 * r + 2]
    gs = [gbuf[r, pl.ds(v * 16, 16)] for v in range(_VPC)]
    m01 = jnp.minimum(gs[0], gs[1])
    m23 = jnp.minimum(gs[2], gs[3])
    m45 = jnp.minimum(gs[4], gs[5])
    m67 = jnp.minimum(gs[6], gs[7])
    m = jnp.minimum(jnp.minimum(m01, m23), jnp.minimum(m45, m67))
    hit = jnp.min(m) < thr

    def drill(bv, bi, thr, r=r):
      def vec(v, st):
        bv, bi, thr = st
        g = gbuf[r, pl.ds(v * 16, 16)]
        vhit = jnp.min(g) < thr

        def do_vec(bv, bi, thr):
          def group(j, st2):
            bv, bi, thr = st2
            vals = pbuf[r, pl.ds(v * 256 + j * 16, 16)]
            ghit = jnp.min(vals) < thr

            def do_merge(bv, bi, thr):
              ci = cbase + v * 256 + j * 16 + iota
              bv, bi = _merge16(bv, bi, vals, ci)
              return bv, bi, bv[15]

            return lax.cond(ghit, do_merge,
                            lambda bv, bi, thr: (bv, bi, thr), bv, bi, thr)

          return lax.fori_loop(0, 16, group, (bv, bi, thr))

        return lax.cond(vhit, do_vec,
                        lambda bv, bi, thr: (bv, bi, thr), bv, bi, thr)

      return lax.fori_loop(0, _VPC, vec, (bv, bi, thr))

    nb = lax.cond(hit, drill, lambda bv, bi, thr: (bv, bi, thr),
                  bv, bi, thr)
    out[3 * r], out[3 * r + 1], out[3 * r + 2] = nb
  return tuple(out)


def _topk_body(proxy, gmin, d2_out, idx_out,
               pbuf0, pbuf1, gbuf0, gbuf1, res_v, resi_v,
               sem_a, sem_b, sem_ga, sem_gb):
  c = lax.axis_index("c")
  s = lax.axis_index("s")
  wid = s * 2 + c
  qbase = wid * _QPW

  def octet(o, _):
    rbase = qbase + o * 8

    def pslab(cb):
      return proxy.at[pl.ds(rbase, 8), pl.ds(cb, _C)]

    def gslab(cb):
      return gmin.at[pl.ds(rbase, 8), pl.ds(cb, _GC)]

    pltpu.async_copy(pslab(0), pbuf0, sem_a)
    pltpu.async_copy(gslab(0), gbuf0, sem_ga)

    init = []
    for _r in range(8):
      init += [jnp.full((16,), jnp.inf, jnp.float32),
               jnp.zeros((16,), jnp.int32), jnp.float32(jnp.inf)]

    def pair(i, carry):
      c0 = 2 * i
      pltpu.async_copy(pslab((c0 + 1) * _C), pbuf1, sem_b)
      pltpu.async_copy(gslab((c0 + 1) * _GC), gbuf1, sem_gb)
      pltpu.make_async_copy(pslab(c0 * _C), pbuf0, sem_a).wait()
      pltpu.make_async_copy(gslab(c0 * _GC), gbuf0, sem_ga).wait()
      carry = _scan_chunk(pbuf0, gbuf0, c0, carry)

      pltpu.async_copy(pslab((c0 + 2) * _C), pbuf0, sem_a)
      pltpu.async_copy(gslab((c0 + 2) * _GC), gbuf0, sem_ga)

      pltpu.make_async_copy(pslab((c0 + 1) * _C), pbuf1, sem_b).wait()
      pltpu.make_async_copy(gslab((c0 + 1) * _GC), gbuf1, sem_gb).wait()
      carry = _scan_chunk(pbuf1, gbuf1, c0 + 1, carry)
      return carry

    carry = lax.fori_loop(0, _NCH // 2, pair, tuple(init))

    last = _NCH - 1
    pltpu.make_async_copy(pslab(last * _C), pbuf0, sem_a).wait()
    pltpu.make_async_copy(gslab(last * _GC), gbuf0, sem_ga).wait()
    carry = _scan_chunk(pbuf0, gbuf0, last, carry)

    for r in range(8):
      res_v[r] = carry[3 * r]
      resi_v[r] = carry[3 * r + 1]
    pltpu.sync_copy(res_v, d2_out.at[pl.ds(rbase, 8)])
    pltpu.sync_copy(resi_v, idx_out.at[pl.ds(rbase, 8)])
    return 0

  lax.fori_loop(0, _QPW // 8, octet, 0)


def _topk(proxy, gmin):
  mesh = plsc.VectorSubcoreMesh(core_axis_name="c", subcore_axis_name="s")
  f = pl.kernel(
      _topk_body,
      out_type=(
          jax.ShapeDtypeStruct((_M, _K), jnp.float32),
          jax.ShapeDtypeStruct((_M, _K), jnp.int32),
      ),
      mesh=mesh,
      scratch_types=[
          pltpu.VMEM((8, _C), jnp.float32),
          pltpu.VMEM((8, _C), jnp.float32),
          pltpu.VMEM((8, _GC), jnp.float32),
          pltpu.VMEM((8, _GC), jnp.float32),
          pltpu.VMEM((8, _K), jnp.float32),
          pltpu.VMEM((8, _K), jnp.int32),
          pltpu.SemaphoreType.DMA,
          pltpu.SemaphoreType.DMA,
          pltpu.SemaphoreType.DMA,
          pltpu.SemaphoreType.DMA,
      ],
      compiler_params=pltpu.CompilerParams(needs_layout_passes=False),
  )
  return f(proxy, gmin)


def _sqrt_body(bv_ref, q_ref, out_ref):
  q = q_ref[...]
  qn = jnp.sum(q * q, axis=1, keepdims=True)
  out_ref[...] = jnp.sqrt(jnp.maximum(bv_ref[...] + qn, 0.0))


def _sqrt(bv, q):
  return pl.pallas_call(
      _sqrt_body,
      out_shape=jax.ShapeDtypeStruct((_M, _K), jnp.float32),
  )(bv, q)


def kernel(query, support):
  q = query[0]
  s = support[0]
  qm2 = -2.0 * q
  spad = jnp.pad(s, ((0, _NPAD - _N), (0, 0)), constant_values=_PADVAL)
  proxy, gmin = _compute_proxy(qm2, spad)
  bv, idx = _topk(proxy, gmin)
  values = _sqrt(bv, q)
  return (values.reshape(1, _M, _K), idx.reshape(1, _M, _K))
